# baseline (speedup 1.0000x reference)
import functools

import jax
import jax.numpy as jnp
from jax import lax
from jax.experimental import pallas as pl
from jax.experimental.pallas import tpu as pltpu
from jax.experimental.pallas import tpu_sc as plsc

VOCAB = 100000
EMBED = 64
BATCH = 4096

NUM_CORES = 2
NUM_SUBCORES = 16
NUM_WORKERS = NUM_CORES * NUM_SUBCORES
B_PER_W = BATCH // NUM_WORKERS
LANES = 16
PACKED = 2 * EMBED

_RB = 12800
HALF = 51200
_NB = HALF // _RB


def _pad_body(wTa_ref, wTb_ref, o_ref):
    o_ref[:, :EMBED] = wTa_ref[...].T
    o_ref[:, EMBED:] = wTb_ref[...].T


@jax.jit
def _pad_one(wT):
    return pl.pallas_call(
        _pad_body,
        grid=(_NB,),
        in_specs=[
            pl.BlockSpec((EMBED, _RB), lambda i: (0, i)),
            pl.BlockSpec((EMBED, _RB), lambda i: (0, i + _NB)),
        ],
        out_specs=pl.BlockSpec((_RB, PACKED), lambda i: (i, 0)),
        out_shape=jax.ShapeDtypeStruct((HALF, PACKED), jnp.float32),
        compiler_params=pltpu.CompilerParams(
            dimension_semantics=("arbitrary",),
        ),
    )(wT, wT)


def _split_idx(idx_slice):
    hi = (idx_slice >= HALF).astype(jnp.int32)
    return idx_slice - hi * HALF, hi * EMBED


def _sc_bias_body(iw_hbm, tw_hbm, b_hbm, bt_hbm, bsum_hbm,
                  idx_i, idx_t, bi_v, bt_v, bsum_v, sem):
    wid = lax.axis_index("s") * NUM_CORES + lax.axis_index("c")
    base = wid * B_PER_W
    pltpu.sync_copy(iw_hbm.at[pl.ds(base, B_PER_W)], idx_i)
    pltpu.sync_copy(tw_hbm.at[pl.ds(base, B_PER_W)], idx_t)
    c0 = pltpu.async_copy(b_hbm.at[idx_i], bi_v, sem)
    c1 = pltpu.async_copy(bt_hbm.at[idx_t], bt_v, sem)
    c0.wait()
    c1.wait()
    for g in range(B_PER_W // LANES):
        s = pl.ds(g * LANES, LANES)
        bsum_v[s] = bi_v[s] + bt_v[s]
    pltpu.sync_copy(bsum_v, bsum_hbm.at[pl.ds(base, B_PER_W)])


_sc_bias = functools.partial(
    pl.kernel,
    out_type=jax.ShapeDtypeStruct((BATCH,), jnp.float32),
    mesh=plsc.VectorSubcoreMesh(core_axis_name="c", subcore_axis_name="s"),
    compiler_params=pltpu.CompilerParams(
        needs_layout_passes=False, use_tc_tiling_on_sc=True),
    scratch_types=[
        pltpu.VMEM((B_PER_W,), jnp.int32),
        pltpu.VMEM((B_PER_W,), jnp.int32),
        pltpu.VMEM((B_PER_W,), jnp.float32),
        pltpu.VMEM((B_PER_W,), jnp.float32),
        pltpu.VMEM((B_PER_W,), jnp.float32),
        pltpu.SemaphoreType.DMA,
    ],
)(_sc_bias_body)


def _sc_gather_e_body(iw_hbm, we_hbm, e_out,
                      idx_i, pidx_i, e_v, sem):
    wid = lax.axis_index("s") * NUM_CORES + lax.axis_index("c")
    base = wid * B_PER_W
    pltpu.sync_copy(iw_hbm.at[pl.ds(base, B_PER_W)], idx_i)
    for g in range(B_PER_W // LANES):
        s = pl.ds(g * LANES, LANES)
        pidx_i[s], _ = _split_idx(idx_i[s])
    pltpu.async_copy(we_hbm.at[pidx_i], e_v, sem).wait()
    pltpu.sync_copy(e_v, e_out.at[pl.ds(base, B_PER_W)])


_sc_gather_e = functools.partial(
    pl.kernel,
    out_type=jax.ShapeDtypeStruct((BATCH, PACKED), jnp.float32),
    mesh=plsc.VectorSubcoreMesh(core_axis_name="c", subcore_axis_name="s"),
    compiler_params=pltpu.CompilerParams(
        needs_layout_passes=False, use_tc_tiling_on_sc=True),
    scratch_types=[
        pltpu.VMEM((B_PER_W,), jnp.int32),
        pltpu.VMEM((B_PER_W,), jnp.int32),
        pltpu.VMEM((B_PER_W, PACKED), jnp.float32),
        pltpu.SemaphoreType.DMA,
    ],
)(_sc_gather_e_body)


def _sc_dot_body(iw_hbm, tw_hbm, wt_hbm, e_rows_hbm, dot_hbm,
                 idx_i, idx_t, pidx_t, off_t, e_v, t_v, dot_v, sem):
    wid = lax.axis_index("s") * NUM_CORES + lax.axis_index("c")
    base = wid * B_PER_W

    pltpu.sync_copy(iw_hbm.at[pl.ds(base, B_PER_W)], idx_i)
    pltpu.sync_copy(tw_hbm.at[pl.ds(base, B_PER_W)], idx_t)

    for g in range(B_PER_W // LANES):
        s = pl.ds(g * LANES, LANES)
        pidx_t[s], off_t[s] = _split_idx(idx_t[s])

    c0 = pltpu.async_copy(wt_hbm.at[pidx_t], t_v, sem)
    c1 = pltpu.async_copy(e_rows_hbm.at[pl.ds(base, B_PER_W)], e_v, sem)
    c0.wait()
    c1.wait()

    lane = lax.iota(jnp.int32, LANES)
    for g in range(B_PER_W // LANES):
        s = pl.ds(g * LANES, LANES)
        row_idx = g * LANES + lane
        oe = (idx_i[s] >= HALF).astype(jnp.int32) * EMBED
        ot = off_t[s]

        def col(c, acc, row_idx=row_idx, oe=oe, ot=ot):
            cb = jnp.full((LANES,), c, jnp.int32)
            ev = plsc.load_gather(e_v, [row_idx, oe + cb])
            tv = plsc.load_gather(t_v, [row_idx, ot + cb])
            return acc + ev * tv

        dot_v[s] = lax.fori_loop(0, EMBED, col, jnp.zeros((LANES,), jnp.float32))

    pltpu.sync_copy(dot_v, dot_hbm.at[pl.ds(base, B_PER_W)])


_sc_dot = functools.partial(
    pl.kernel,
    out_type=jax.ShapeDtypeStruct((BATCH,), jnp.float32),
    mesh=plsc.VectorSubcoreMesh(core_axis_name="c", subcore_axis_name="s"),
    compiler_params=pltpu.CompilerParams(
        needs_layout_passes=False, use_tc_tiling_on_sc=True),
    scratch_types=[
        pltpu.VMEM((B_PER_W,), jnp.int32),
        pltpu.VMEM((B_PER_W,), jnp.int32),
        pltpu.VMEM((B_PER_W,), jnp.int32),
        pltpu.VMEM((B_PER_W,), jnp.int32),
        pltpu.VMEM((B_PER_W, PACKED), jnp.float32),
        pltpu.VMEM((B_PER_W, PACKED), jnp.float32),
        pltpu.VMEM((B_PER_W,), jnp.float32),
        pltpu.SemaphoreType.DMA,
    ],
)(_sc_dot_body)


def _tc_body(bsum_ref, dot_ref, out_ref):
    out_ref[...] = bsum_ref[...] + dot_ref[...]


_BM = 512


@jax.jit
def _broadcast_add(bsum, dot):
    return pl.pallas_call(
        _tc_body,
        grid=(BATCH // _BM,),
        in_specs=[
            pl.BlockSpec((_BM, 1), lambda i: (i, 0)),
            pl.BlockSpec((1, BATCH), lambda i: (0, 0)),
        ],
        out_specs=pl.BlockSpec((_BM, BATCH), lambda i: (i, 0)),
        out_shape=jax.ShapeDtypeStruct((BATCH, BATCH), jnp.float32),
        compiler_params=pltpu.CompilerParams(
            dimension_semantics=("arbitrary",),
        ),
    )(bsum, dot)


@jax.jit
def kernel(input_word, target_word, W_embed, W_tilda, b, b_tilda):
    iw = input_word.astype(jnp.int32)
    tw = target_word.astype(jnp.int32)
    bsum = _sc_bias(iw, tw, jnp.sum(b, axis=1), jnp.sum(b_tilda, axis=1))
    we2 = _pad_one(W_embed.T)
    e_rows = _sc_gather_e(iw, we2)
    wt2 = _pad_one(W_tilda.T)
    dot = _sc_dot(iw, tw, wt2, e_rows)
    return _broadcast_add(bsum.reshape(BATCH, 1), dot.reshape(1, BATCH))


# speedup vs baseline: 1.0494x; 1.0494x over previous
import functools

import jax
import jax.numpy as jnp
from jax import lax
from jax.experimental import pallas as pl
from jax.experimental.pallas import tpu as pltpu
from jax.experimental.pallas import tpu_sc as plsc

VOCAB = 100000
EMBED = 64
BATCH = 4096

NUM_CORES = 2
NUM_SUBCORES = 16
NUM_WORKERS = NUM_CORES * NUM_SUBCORES
B_PER_W = BATCH // NUM_WORKERS
LANES = 16
PACKED = 2 * EMBED

_RB = 6400
HALF = 51200
_NB = HALF // _RB


def _pad_body(weTa_ref, weTb_ref, wtTa_ref, wtTb_ref, oe_ref, ot_ref):
    oe_ref[:, :EMBED] = weTa_ref[...].T
    oe_ref[:, EMBED:] = weTb_ref[...].T
    ot_ref[:, :EMBED] = wtTa_ref[...].T
    ot_ref[:, EMBED:] = wtTb_ref[...].T


@jax.jit
def _pad_tables(weT, wtT):
    return pl.pallas_call(
        _pad_body,
        grid=(_NB,),
        in_specs=[
            pl.BlockSpec((EMBED, _RB), lambda i: (0, i)),
            pl.BlockSpec((EMBED, _RB), lambda i: (0, i + _NB)),
            pl.BlockSpec((EMBED, _RB), lambda i: (0, i)),
            pl.BlockSpec((EMBED, _RB), lambda i: (0, i + _NB)),
        ],
        out_specs=[
            pl.BlockSpec((_RB, PACKED), lambda i: (i, 0)),
            pl.BlockSpec((_RB, PACKED), lambda i: (i, 0)),
        ],
        out_shape=(
            jax.ShapeDtypeStruct((HALF, PACKED), jnp.float32),
            jax.ShapeDtypeStruct((HALF, PACKED), jnp.float32),
        ),
        compiler_params=pltpu.CompilerParams(
            dimension_semantics=("arbitrary",),
        ),
    )(weT, weT, wtT, wtT)


def _sc_bias_body(iw_hbm, tw_hbm, b_hbm, bt_hbm, bsum_hbm,
                  idx_i, idx_t, bi_v, bt_v, bsum_v, sem):
    wid = lax.axis_index("s") * NUM_CORES + lax.axis_index("c")
    base = wid * B_PER_W
    pltpu.sync_copy(iw_hbm.at[pl.ds(base, B_PER_W)], idx_i)
    pltpu.sync_copy(tw_hbm.at[pl.ds(base, B_PER_W)], idx_t)
    c0 = pltpu.async_copy(b_hbm.at[idx_i], bi_v, sem)
    c1 = pltpu.async_copy(bt_hbm.at[idx_t], bt_v, sem)
    c0.wait()
    c1.wait()
    for g in range(B_PER_W // LANES):
        s = pl.ds(g * LANES, LANES)
        bsum_v[s] = bi_v[s] + bt_v[s]
    pltpu.sync_copy(bsum_v, bsum_hbm.at[pl.ds(base, B_PER_W)])


_sc_bias = functools.partial(
    pl.kernel,
    out_type=jax.ShapeDtypeStruct((BATCH,), jnp.float32),
    mesh=plsc.VectorSubcoreMesh(core_axis_name="c", subcore_axis_name="s"),
    compiler_params=pltpu.CompilerParams(
        needs_layout_passes=False, use_tc_tiling_on_sc=True),
    scratch_types=[
        pltpu.VMEM((B_PER_W,), jnp.int32),
        pltpu.VMEM((B_PER_W,), jnp.int32),
        pltpu.VMEM((B_PER_W,), jnp.float32),
        pltpu.VMEM((B_PER_W,), jnp.float32),
        pltpu.VMEM((B_PER_W,), jnp.float32),
        pltpu.SemaphoreType.DMA,
    ],
)(_sc_bias_body)


def _sc_body(iw_hbm, tw_hbm, we_hbm, wt_hbm,
             dot_hbm,
             idx_i, idx_t, pidx_i, pidx_t, off_i, off_t, e_v, t_v,
             dot_v, sem):
    wid = lax.axis_index("s") * NUM_CORES + lax.axis_index("c")
    base = wid * B_PER_W

    pltpu.sync_copy(iw_hbm.at[pl.ds(base, B_PER_W)], idx_i)
    pltpu.sync_copy(tw_hbm.at[pl.ds(base, B_PER_W)], idx_t)

    for g in range(B_PER_W // LANES):
        s = pl.ds(g * LANES, LANES)
        hi = (idx_i[s] >= HALF).astype(jnp.int32)
        ht = (idx_t[s] >= HALF).astype(jnp.int32)
        pidx_i[s] = idx_i[s] - hi * HALF
        pidx_t[s] = idx_t[s] - ht * HALF
        off_i[s] = hi * EMBED
        off_t[s] = ht * EMBED

    c0 = pltpu.async_copy(we_hbm.at[pidx_i], e_v, sem)
    c1 = pltpu.async_copy(wt_hbm.at[pidx_t], t_v, sem)
    c0.wait()
    c1.wait()

    lane = lax.iota(jnp.int32, LANES)
    for g in range(B_PER_W // LANES):
        s = pl.ds(g * LANES, LANES)
        row_idx = g * LANES + lane
        oe = off_i[s]
        ot = off_t[s]

        def col(c, acc, row_idx=row_idx, oe=oe, ot=ot):
            cb = jnp.full((LANES,), c, jnp.int32)
            ev = plsc.load_gather(e_v, [row_idx, oe + cb])
            tv = plsc.load_gather(t_v, [row_idx, ot + cb])
            return acc + ev * tv

        dot_v[s] = lax.fori_loop(0, EMBED, col, jnp.zeros((LANES,), jnp.float32), unroll=8)

    pltpu.sync_copy(dot_v, dot_hbm.at[pl.ds(base, B_PER_W)])


_sc_gather_dot = functools.partial(
    pl.kernel,
    out_type=jax.ShapeDtypeStruct((BATCH,), jnp.float32),
    mesh=plsc.VectorSubcoreMesh(core_axis_name="c", subcore_axis_name="s"),
    compiler_params=pltpu.CompilerParams(
        needs_layout_passes=False, use_tc_tiling_on_sc=True),
    scratch_types=[
        pltpu.VMEM((B_PER_W,), jnp.int32),
        pltpu.VMEM((B_PER_W,), jnp.int32),
        pltpu.VMEM((B_PER_W,), jnp.int32),
        pltpu.VMEM((B_PER_W,), jnp.int32),
        pltpu.VMEM((B_PER_W,), jnp.int32),
        pltpu.VMEM((B_PER_W,), jnp.int32),
        pltpu.VMEM((B_PER_W, PACKED), jnp.float32),
        pltpu.VMEM((B_PER_W, PACKED), jnp.float32),
        pltpu.VMEM((B_PER_W,), jnp.float32),
        pltpu.SemaphoreType.DMA,
    ],
)(_sc_body)


def _tc_body(bsum_ref, dot_ref, out_ref):
    out_ref[...] = bsum_ref[...] + dot_ref[...]


_BM = 512


@jax.jit
def _broadcast_add(bsum, dot):
    return pl.pallas_call(
        _tc_body,
        grid=(BATCH // _BM,),
        in_specs=[
            pl.BlockSpec((_BM, 1), lambda i: (i, 0)),
            pl.BlockSpec((1, BATCH), lambda i: (0, 0)),
        ],
        out_specs=pl.BlockSpec((_BM, BATCH), lambda i: (i, 0)),
        out_shape=jax.ShapeDtypeStruct((BATCH, BATCH), jnp.float32),
        compiler_params=pltpu.CompilerParams(
            dimension_semantics=("arbitrary",),
        ),
    )(bsum, dot)


@jax.jit
def kernel(input_word, target_word, W_embed, W_tilda, b, b_tilda):
    iw = input_word.astype(jnp.int32)
    tw = target_word.astype(jnp.int32)
    bsum = _sc_bias(iw, tw, jnp.sum(b, axis=1), jnp.sum(b_tilda, axis=1))
    we2, wt2 = _pad_tables(W_embed.T, W_tilda.T)
    dot = _sc_gather_dot(iw, tw, we2, wt2)
    return _broadcast_add(bsum.reshape(BATCH, 1), dot.reshape(1, BATCH))
